# baseline (speedup 1.0000x reference)
import functools

import jax
import jax.numpy as jnp
from jax import lax
from jax.experimental import pallas as pl
from jax.experimental.pallas import tpu as pltpu
from jax.experimental.pallas import tpu_sc as plsc

_H = 224
_W = 224
_STRIDE = 8
_PATCH = 16
_B = 4
_C = 32
_NY = 27
_NX = 27
_PXC = _PATCH * _C
_UNITS = 28
_ROWS_PER_WORKER = (_B * _H) // 32

_mesh = plsc.VectorSubcoreMesh(core_axis_name="c", subcore_axis_name="s")


@functools.partial(
    pl.kernel,
    out_type=jax.ShapeDtypeStruct((_B, _H, _C, _W), jnp.float32),
    mesh=_mesh,
    scratch_types=[
        pltpu.VMEM((2, _NX, _PXC), jnp.float32),
        pltpu.VMEM((2, _NX, _PXC), jnp.float32),
        pltpu.VMEM((2, _C, _W), jnp.float32),
        pltpu.SemaphoreType.DMA((2,)),
        pltpu.SemaphoreType.DMA((2,)),
        pltpu.SemaphoreType.DMA((2,)),
    ],
    compiler_params=pltpu.CompilerParams(needs_layout_passes=False),
)
def _overlap_add_sc(x_ref, out_ref, pa, pb, ob, sema, semb, semo):
    c_iota = lax.iota(jnp.int32, 16)
    cid = lax.axis_index("c")
    sid = lax.axis_index("s")
    wid = cid * 16 + sid
    bidx = wid // 8
    h0 = (wid % 8) * _ROWS_PER_WORKER

    def in_copies(row, j):
        h = h0 + row
        iy_a = jnp.minimum(lax.div(h, 8), _NY - 1)
        py_a = h - 8 * iy_a
        iy_b = jnp.maximum(iy_a - 1, 0)
        py_b = jnp.minimum(py_a + 8, _PATCH - 1)
        ca = pltpu.make_async_copy(
            x_ref.at[bidx, iy_a, :, pl.ds(py_a * _PXC, _PXC)],
            pa.at[j], sema.at[j])
        cb = pltpu.make_async_copy(
            x_ref.at[bidx, iy_b, :, pl.ds(py_b * _PXC, _PXC)],
            pb.at[j], semb.at[j])
        return ca, cb

    def out_copy(row, j):
        h = h0 + row
        return pltpu.make_async_copy(ob.at[j], out_ref.at[bidx, h], semo.at[j])

    ca0, cb0 = in_copies(0, 0)
    ca0.start()
    cb0.start()

    def row_body(i, carry):
        j = lax.rem(i, 2)
        h = h0 + i

        @pl.when(i + 1 < _ROWS_PER_WORKER)
        def _prefetch():
            ca, cb = in_copies(i + 1, 1 - j)
            ca.start()
            cb.start()

        ca, cb = in_copies(i, j)
        ca.wait()
        cb.wait()

        two_y = jnp.logical_and(h >= _STRIDE, h < _H - _STRIDE)
        wy = jnp.where(two_y, 1.0, 0.0).astype(jnp.float32)
        s05 = jnp.where(two_y, 0.25, 0.5).astype(jnp.float32)
        wyv = jnp.full((16,), wy, jnp.float32)
        s05v = jnp.full((16,), s05, jnp.float32)
        syv = s05v + s05v

        @pl.when(i >= 2)
        def _drain_out():
            out_copy(i, j).wait()

        obj = ob.at[j]

        def put(val, w0, v):
            plsc.store_scatter(
                obj, [c_iota + 16 * (v % 2), jnp.full((16,), w0, jnp.int32)],
                val)

        for v in range(16):
            a = pa[j, 0, pl.ds(v * 16, 16)]
            b = pb[j, 0, pl.ds(v * 16, 16)]
            put((a + wyv * b) * syv, v // 2, v)
            a = pa[j, _NX - 1, pl.ds(256 + v * 16, 16)]
            b = pb[j, _NX - 1, pl.ds(256 + v * 16, 16)]
            put((a + wyv * b) * syv, 27 * 8 + v // 2, v)

        def unit_body(r, c2):
            for v in range(16):
                a = pa[j, r, pl.ds(v * 16, 16)] + pa[j, r - 1, pl.ds(256 + v * 16, 16)]
                b = pb[j, r, pl.ds(v * 16, 16)] + pb[j, r - 1, pl.ds(256 + v * 16, 16)]
                put((a + wyv * b) * s05v, r * 8 + v // 2, v)
            return c2

        lax.fori_loop(1, _UNITS - 1, unit_body, 0)
        out_copy(i, j).start()
        return carry

    lax.fori_loop(0, _ROWS_PER_WORKER, row_body, 0)

    out_copy(_ROWS_PER_WORKER - 2, lax.rem(_ROWS_PER_WORKER - 2, 2)).wait()
    out_copy(_ROWS_PER_WORKER - 1, lax.rem(_ROWS_PER_WORKER - 1, 2)).wait()


def kernel(x):
    xr = x.reshape(_B, _NY, _NX, _PATCH * _PXC)
    out = _overlap_add_sc(xr)
    return jnp.transpose(out, (0, 1, 3, 2))


# speedup vs baseline: 2.2593x; 2.2593x over previous
import functools

import numpy as np

import jax
import jax.numpy as jnp
from jax import lax
from jax.experimental import pallas as pl
from jax.experimental.pallas import tpu as pltpu
from jax.experimental.pallas import tpu_sc as plsc

_H = 224
_W = 224
_B = 4
_C = 32
_NY = 27
_NX = 27
_NP = _NY * _NX
_ROWSZ = _C * _W
_NSLOT = 8
_NTILE = 6


def _run_specs():
    specs = []
    for t in range(5):
        for v in range(8):
            specs.append((t, 16 * v, 0))
    for v in range(5):
        specs.append((5, 16 * v, 0))
    specs.append((5, 73, 7))
    return specs


_SPECS = _run_specs()


def _base_tables():
    a, bb, masks = [], [], []
    for (t, col0, mfrom) in _SPECS:
        p = 128 * t + col0 + np.arange(16)
        p = np.minimum(p, _NP - 1)
        iy = p // _NX
        ix = p % _NX
        a.append(((iy % _NSLOT) * _ROWSZ + 8 * ix).astype(np.int32))
        bb.append((((iy + 1) % _NSLOT) * _ROWSZ + 8 * ix).astype(np.int32))
        masks.append((np.arange(16) >= mfrom))
    return np.stack(a), np.stack(bb), np.stack(masks)


_BASE_A, _BASE_B, _MASKS = _base_tables()

_DONE_AT = [[0, 1, 2, 3], [4, 5, 6, 7, 8], [9, 10, 11, 12, 13],
            [14, 15, 16, 17], [18, 19, 20, 21, 22], [23, 24, 25, 26, 27]]

_mesh = plsc.VectorSubcoreMesh(core_axis_name="c", subcore_axis_name="s")


@functools.partial(
    pl.kernel,
    out_type=jax.ShapeDtypeStruct((_B, _H, _C, _W), jnp.float32),
    mesh=_mesh,
    scratch_types=[
        pltpu.VMEM((2, 2, _C, 128), jnp.float32),
        pltpu.VMEM((_NSLOT * _ROWSZ,), jnp.float32),
        pltpu.VMEM((2, _C, _W), jnp.float32),
        pltpu.VMEM((len(_SPECS) * 2 * 16,), jnp.int32),
        pltpu.SemaphoreType.DMA((2,)),
        pltpu.SemaphoreType.DMA((2,)),
        pltpu.SemaphoreType.DMA((2,)),
        pltpu.SemaphoreType.DMA,
    ],
    compiler_params=pltpu.CompilerParams(needs_layout_passes=False),
)
def _overlap_add_sc(xt_ref, xtail_ref, tbl_ref, out_ref, tb, acc, stg, tblv,
                    sa, sb, so, st):
    iota16 = lax.iota(jnp.int32, 16)
    cid = lax.axis_index("c")
    sid = lax.axis_index("s")
    wid = cid * 16 + sid
    b = wid // 8
    py = wid % 8

    pltpu.make_async_copy(tbl_ref, tblv, st).start()

    def zbody(i, c2):
        acc[pl.ds(i * 16, 16)] = jnp.zeros((16,), jnp.float32)
        return c2
    lax.fori_loop(0, _NSLOT * _ROWSZ // 16, zbody, 0)
    pltpu.make_async_copy(tbl_ref, tblv, st).wait()

    def in_copies(t, px, j):
        if t == 5:
            srca = xtail_ref.at[b, py, px]
            srcb = xtail_ref.at[b, py + 8, px]
        else:
            srca = xt_ref.at[b, py, px, :, pl.ds(128 * t, 128)]
            srcb = xt_ref.at[b, py + 8, px, :, pl.ds(128 * t, 128)]
        ca = pltpu.make_async_copy(srca, tb.at[0, j], sa.at[j])
        cb = pltpu.make_async_copy(srcb, tb.at[1, j], sb.at[j])
        return ca, cb

    ca, cb = in_copies(0, 0, 0)
    ca.start()
    cb.start()

    run_of_t = [[i for i, s in enumerate(_SPECS) if s[0] == t]
                for t in range(_NTILE)]

    out_row_count = [0]

    def complete_row(r):
        jo = out_row_count[0] % 2
        out_row_count[0] += 1
        slot = r % _NSLOT
        sy = 1.0 if (r == 0 or r == 27) else 0.5
        mid = jnp.full((16,), sy * 0.5, jnp.float32)
        lo = jnp.where(iota16 < 8, sy, sy * 0.5).astype(jnp.float32)
        hi = jnp.where(iota16 < 8, sy * 0.5, sy).astype(jnp.float32)

        if out_row_count[0] > 2:
            pltpu.make_async_copy(stg.at[jo], out_ref.at[b, 0], so.at[jo]).wait()

        def crow(c, c2):
            base = slot * _ROWSZ + c * _W
            for k in range(14):
                scale = lo if k == 0 else (hi if k == 13 else mid)
                v = acc[pl.ds(base + 16 * k, 16)] * scale
                stg[jo, c, pl.ds(16 * k, 16)] = v
                acc[pl.ds(base + 16 * k, 16)] = jnp.zeros((16,), jnp.float32)
            return c2
        lax.fori_loop(0, _C, crow, 0)
        h = 8 * r + py
        pltpu.make_async_copy(stg.at[jo], out_ref.at[b, h], so.at[jo]).start()

    for t in range(_NTILE):
        def px_body(px, c2):
            j = lax.rem(px, 2)

            @pl.when(px + 1 < 16)
            def _pref():
                ca, cb = in_copies(t, px + 1, 1 - j)
                ca.start()
                cb.start()

            ca, cb = in_copies(t, px, j)
            ca.wait()
            cb.wait()

            def c_body(c, c3):
                off = c * _W + px
                for plane in range(2):
                    for ri in run_of_t[t]:
                        tv = tblv[pl.ds(16 * (2 * ri + plane), 16)]
                        idx = tv + jnp.full((16,), off, jnp.int32)
                        val = tb[plane, j, c, pl.ds(_SPECS[ri][1], 16)]
                        if _SPECS[ri][2]:
                            plsc.addupdate_scatter(
                                acc, [idx], val,
                                mask=(iota16 >= _SPECS[ri][2]))
                        else:
                            plsc.addupdate_scatter(acc, [idx], val)
                return c3
            lax.fori_loop(0, _C, c_body, 0)
            return c2
        lax.fori_loop(0, 16, px_body, 0)

        if t + 1 < _NTILE:
            ca, cb = in_copies(t + 1, 0, 0)
            ca.start()
            cb.start()

        for r in _DONE_AT[t]:
            complete_row(r)

    n = out_row_count[0]
    pltpu.make_async_copy(stg.at[(n - 2) % 2], out_ref.at[b, 0],
                          so.at[(n - 2) % 2]).wait()
    pltpu.make_async_copy(stg.at[(n - 1) % 2], out_ref.at[b, 0],
                          so.at[(n - 1) % 2]).wait()


def kernel(x):
    xt = jnp.transpose(x, (0, 2, 3, 4, 1))
    xtail = jnp.pad(xt[:, :, :, :, 640:], ((0, 0),) * 4 + ((0, 39),))
    tbl = jnp.asarray(
        np.stack([np.stack([_BASE_A[i], _BASE_B[i]])
                  for i in range(len(_SPECS))]).reshape(-1))
    out = _overlap_add_sc(xt, xtail, tbl)
    return jnp.transpose(out, (0, 1, 3, 2))


# speedup vs baseline: 2.3855x; 1.0559x over previous
import functools

import numpy as np

import jax
import jax.numpy as jnp
from jax import lax
from jax.experimental import pallas as pl
from jax.experimental.pallas import tpu as pltpu
from jax.experimental.pallas import tpu_sc as plsc

_H = 224
_W = 224
_B = 4
_C = 32
_NY = 27
_NX = 27
_NP = _NY * _NX
_NSLOT = 8
_NTILE = 6
_QSTR = _C * 8 + 1
_SLOTSZ = 7200


def _run_specs():
    specs = []
    for t in range(5):
        for v in range(8):
            specs.append((t, 16 * v, 0))
    for v in range(5):
        specs.append((5, 16 * v, 0))
    specs.append((5, 73, 7))
    return specs


_SPECS = _run_specs()


def _base_tables():
    a, bb, masks = [], [], []
    for (t, col0, mfrom) in _SPECS:
        p = 128 * t + col0 + np.arange(16)
        p = np.minimum(p, _NP - 1)
        iy = p // _NX
        ix = p % _NX
        a.append(((iy % _NSLOT) * _SLOTSZ + _QSTR * ix).astype(np.int32))
        bb.append((((iy + 1) % _NSLOT) * _SLOTSZ + _QSTR * ix).astype(np.int32))
        masks.append((np.arange(16) >= mfrom))
    return np.stack(a), np.stack(bb), np.stack(masks)


_BASE_A, _BASE_B, _MASKS = _base_tables()

_DONE_AT = [[0, 1, 2, 3], [4, 5, 6, 7, 8], [9, 10, 11, 12, 13],
            [14, 15, 16, 17], [18, 19, 20, 21, 22], [23, 24, 25, 26, 27]]

_mesh = plsc.VectorSubcoreMesh(core_axis_name="c", subcore_axis_name="s")


@functools.partial(
    pl.kernel,
    out_type=jax.ShapeDtypeStruct((_B, _H, _C, _W), jnp.float32),
    mesh=_mesh,
    scratch_types=[
        pltpu.VMEM((2, 2, _C, 128), jnp.float32),
        pltpu.VMEM((_NSLOT * _SLOTSZ,), jnp.float32),
        pltpu.VMEM((2, _C, _W), jnp.float32),
        pltpu.VMEM((len(_SPECS) * 2 * 16,), jnp.int32),
        pltpu.SemaphoreType.DMA((2,)),
        pltpu.SemaphoreType.DMA((2,)),
        pltpu.SemaphoreType.DMA((2,)),
        pltpu.SemaphoreType.DMA,
    ],
    compiler_params=pltpu.CompilerParams(needs_layout_passes=False),
)
def _overlap_add_sc(xt_ref, xtail_ref, tbl_ref, out_ref, tb, acc, stg, tblv,
                    sa, sb, so, st):
    iota16 = lax.iota(jnp.int32, 16)
    patt = lax.div(iota16, 8) * _QSTR + lax.rem(iota16, 8)
    cid = lax.axis_index("c")
    sid = lax.axis_index("s")
    wid = cid * 16 + sid
    b = wid // 8
    py = wid % 8

    pltpu.make_async_copy(tbl_ref, tblv, st).start()

    def zbody(i, c2):
        acc[pl.ds(i * 16, 16)] = jnp.zeros((16,), jnp.float32)
        return c2
    lax.fori_loop(0, _NSLOT * _SLOTSZ // 16, zbody, 0)
    pltpu.make_async_copy(tbl_ref, tblv, st).wait()

    def in_copies(t, px, j):
        if t == 5:
            srca = xtail_ref.at[b, py, px]
            srcb = xtail_ref.at[b, py + 8, px]
        else:
            srca = xt_ref.at[b, py, px, :, pl.ds(128 * t, 128)]
            srcb = xt_ref.at[b, py + 8, px, :, pl.ds(128 * t, 128)]
        ca = pltpu.make_async_copy(srca, tb.at[0, j], sa.at[j])
        cb = pltpu.make_async_copy(srcb, tb.at[1, j], sb.at[j])
        return ca, cb

    ca, cb = in_copies(0, 0, 0)
    ca.start()
    cb.start()

    run_of_t = [[i for i, s in enumerate(_SPECS) if s[0] == t]
                for t in range(_NTILE)]

    out_row_count = [0]

    def complete_row(r):
        jo = out_row_count[0] % 2
        out_row_count[0] += 1
        slot = r % _NSLOT
        sy = 1.0 if (r == 0 or r == 27) else 0.5
        mid = jnp.full((16,), sy * 0.5, jnp.float32)
        lo = jnp.where(iota16 < 8, sy, sy * 0.5).astype(jnp.float32)
        hi = jnp.where(iota16 < 8, sy * 0.5, sy).astype(jnp.float32)

        if out_row_count[0] > 2:
            pltpu.make_async_copy(stg.at[jo], out_ref.at[b, 0], so.at[jo]).wait()

        def crow(c, c2):
            base = slot * _SLOTSZ + 8 * c
            for k in range(14):
                scale = lo if k == 0 else (hi if k == 13 else mid)
                idx = patt + jnp.full((16,), base + 2 * _QSTR * k, jnp.int32)
                v = plsc.load_gather(acc, [idx]) * scale
                stg[jo, c, pl.ds(16 * k, 16)] = v
            return c2
        lax.fori_loop(0, _C, crow, 0)

        def zrow(i, c2):
            acc[pl.ds(slot * _SLOTSZ + i * 16, 16)] = jnp.zeros(
                (16,), jnp.float32)
            return c2
        lax.fori_loop(0, _SLOTSZ // 16, zrow, 0)
        h = 8 * r + py
        pltpu.make_async_copy(stg.at[jo], out_ref.at[b, h], so.at[jo]).start()

    for t in range(_NTILE):
        def px_body(px, c2):
            j = lax.rem(px, 2)

            @pl.when(px + 1 < 16)
            def _pref():
                ca, cb = in_copies(t, px + 1, 1 - j)
                ca.start()
                cb.start()

            ca, cb = in_copies(t, px, j)
            ca.wait()
            cb.wait()

            def c_body(c, c3):
                off = _QSTR * lax.div(px, 8) + 8 * c + lax.rem(px, 8)
                for plane in range(2):
                    for ri in run_of_t[t]:
                        tv = tblv[pl.ds(16 * (2 * ri + plane), 16)]
                        idx = tv + jnp.full((16,), off, jnp.int32)
                        val = tb[plane, j, c, pl.ds(_SPECS[ri][1], 16)]
                        if _SPECS[ri][2]:
                            plsc.addupdate_scatter(
                                acc, [idx], val,
                                mask=(iota16 >= _SPECS[ri][2]))
                        else:
                            plsc.addupdate_scatter(acc, [idx], val)
                return c3
            lax.fori_loop(0, _C, c_body, 0)
            return c2
        lax.fori_loop(0, 16, px_body, 0)

        if t + 1 < _NTILE:
            ca, cb = in_copies(t + 1, 0, 0)
            ca.start()
            cb.start()

        for r in _DONE_AT[t]:
            complete_row(r)

    n = out_row_count[0]
    pltpu.make_async_copy(stg.at[(n - 2) % 2], out_ref.at[b, 0],
                          so.at[(n - 2) % 2]).wait()
    pltpu.make_async_copy(stg.at[(n - 1) % 2], out_ref.at[b, 0],
                          so.at[(n - 1) % 2]).wait()


def kernel(x):
    xt = jnp.transpose(x, (0, 2, 3, 4, 1))
    xtail = jnp.pad(xt[:, :, :, :, 640:], ((0, 0),) * 4 + ((0, 39),))
    tbl = jnp.asarray(
        np.stack([np.stack([_BASE_A[i], _BASE_B[i]])
                  for i in range(len(_SPECS))]).reshape(-1))
    out = _overlap_add_sc(xt, xtail, tbl)
    return jnp.transpose(out, (0, 1, 3, 2))


# speedup vs baseline: 7.4281x; 3.1138x over previous
import functools

import numpy as np

import jax
import jax.numpy as jnp
from jax import lax
from jax.experimental import pallas as pl
from jax.experimental.pallas import tpu as pltpu
from jax.experimental.pallas import tpu_sc as plsc

_H = 224
_W = 224
_B = 4
_C = 32
_NY = 27
_NX = 27
_NP = _NY * _NX
_NSLOT = 8
_NTILE = 6
_QSTR = _C * 8 + 1
_SLOTSZ = 7200


def _run_specs():
    specs = []
    for t in range(5):
        for v in range(8):
            specs.append((t, 16 * v, 0))
    for v in range(5):
        specs.append((5, 16 * v, 0))
    specs.append((5, 73, 7))
    return specs


_SPECS = _run_specs()


def _base_tables():
    a, bb = [], []
    for (t, col0, mfrom) in _SPECS:
        p = 128 * t + col0 + np.arange(16)
        p = np.minimum(p, _NP - 1)
        iy = p // _NX
        ix = p % _NX
        a.append(((iy % _NSLOT) * _SLOTSZ + _QSTR * ix).astype(np.int32))
        bb.append((((iy + 1) % _NSLOT) * _SLOTSZ + _QSTR * ix).astype(np.int32))
    return np.stack(a), np.stack(bb)


_BASE_A, _BASE_B = _base_tables()

_mesh = plsc.VectorSubcoreMesh(core_axis_name="c", subcore_axis_name="s")


@functools.partial(
    pl.kernel,
    out_type=jax.ShapeDtypeStruct((_B, _H, _C, _W), jnp.float32),
    mesh=_mesh,
    scratch_types=[
        pltpu.VMEM((2, 2, _C, 128), jnp.float32),
        pltpu.VMEM((_NSLOT * _SLOTSZ,), jnp.float32),
        pltpu.VMEM((2, _C, _W), jnp.float32),
        pltpu.VMEM((len(_SPECS) * 2 * 16,), jnp.int32),
        pltpu.SemaphoreType.DMA((2,)),
        pltpu.SemaphoreType.DMA((2,)),
        pltpu.SemaphoreType.DMA((2,)),
        pltpu.SemaphoreType.DMA,
    ],
    compiler_params=pltpu.CompilerParams(needs_layout_passes=False),
)
def _overlap_add_sc(xt_ref, xtail_ref, tbl_ref, out_ref, tb, acc, stg, tblv,
                    sa, sb, so, st):
    iota16 = lax.iota(jnp.int32, 16)
    patt = lax.div(iota16, 8) * _QSTR + lax.rem(iota16, 8)
    cid = lax.axis_index("c")
    sid = lax.axis_index("s")
    wid = cid * 16 + sid
    b = wid // 8
    py = wid % 8

    pltpu.make_async_copy(tbl_ref, tblv, st).start()

    @plsc.parallel_loop(0, _NSLOT * _SLOTSZ // 16, unroll=4)
    def _zbody(i):
        acc[pl.ds(i * 16, 16)] = jnp.zeros((16,), jnp.float32)

    pltpu.make_async_copy(tbl_ref, tblv, st).wait()

    def in_copies(s, j):
        t = lax.div(s, 16)
        px = lax.rem(s, 16)
        l0 = pl.multiple_of(128 * t, 128)
        srca = xt_ref.at[b, py, px, :, pl.ds(l0, 128)]
        srcb = xt_ref.at[b, py + 8, px, :, pl.ds(l0, 128)]
        ca = pltpu.make_async_copy(srca, tb.at[0, j], sa.at[j])
        cb = pltpu.make_async_copy(srcb, tb.at[1, j], sb.at[j])
        return ca, cb

    def tail_copies(px, j):
        ca = pltpu.make_async_copy(
            xtail_ref.at[b, py, px], tb.at[0, j], sa.at[j])
        cb = pltpu.make_async_copy(
            xtail_ref.at[b, py + 8, px], tb.at[1, j], sb.at[j])
        return ca, cb

    def complete_row(r):
        jo = lax.rem(r, 2)
        slot = lax.rem(r, _NSLOT)
        h = 8 * r + py
        sy = jnp.where(jnp.logical_and(r >= 1, r <= 26), 0.5, 1.0).astype(
            jnp.float32)
        mid = jnp.full((16,), sy * 0.5, jnp.float32)
        lo = jnp.where(iota16 < 8, sy, sy * 0.5).astype(jnp.float32)
        hi = jnp.where(iota16 < 8, sy * 0.5, sy).astype(jnp.float32)

        @pl.when(r >= 2)
        def _drain():
            pltpu.make_async_copy(stg.at[jo], out_ref.at[b, 0],
                                  so.at[jo]).wait()

        @plsc.parallel_loop(0, _C, unroll=2)
        def _crow(c):
            base = slot * _SLOTSZ + 8 * c
            for k in range(14):
                scale = lo if k == 0 else (hi if k == 13 else mid)
                idx = patt + jnp.full((16,), base + 2 * _QSTR * k, jnp.int32)
                v = plsc.load_gather(acc, [idx]) * scale
                stg[jo, c, pl.ds(16 * k, 16)] = v

        pltpu.make_async_copy(stg.at[jo], out_ref.at[b, h], so.at[jo]).start()

        @plsc.parallel_loop(0, _SLOTSZ // 16, unroll=4)
        def _zrow(i):
            acc[pl.ds(slot * _SLOTSZ + i * 16, 16)] = jnp.zeros(
                (16,), jnp.float32)

    def scatter_runs(px, j, runs):
        @plsc.parallel_loop(0, _C)
        def _c_body(c):
            off = _QSTR * lax.div(px, 8) + 8 * c + lax.rem(px, 8)
            for plane in range(2):
                for (tbl_ds, col0, mfrom) in runs:
                    tv = tblv[pl.ds(tbl_ds + 16 * plane, 16)]
                    idx = tv + jnp.full((16,), off, jnp.int32)
                    val = tb[plane, j, c, pl.ds(col0, 16)]
                    if mfrom:
                        plsc.addupdate_scatter(acc, [idx], val,
                                               mask=(iota16 >= mfrom))
                    else:
                        plsc.addupdate_scatter(acc, [idx], val)

    ca, cb = in_copies(0, 0)
    ca.start()
    cb.start()

    def s_body(s, carry):
        j = lax.rem(s, 2)

        @pl.when(s + 1 < 80)
        def _pref():
            ca, cb = in_copies(s + 1, 1 - j)
            ca.start()
            cb.start()

        ca, cb = in_copies(s, j)
        ca.wait()
        cb.wait()

        t = lax.div(s, 16)
        px = lax.rem(s, 16)
        runs = [(32 * (8 * t + v), 16 * v, 0) for v in range(8)]
        scatter_runs(px, j, runs)

        @pl.when(lax.rem(s, 16) == 15)
        def _complete():
            done_prev = jnp.where(t > 0, lax.div(128 * (t - 1) + 101, 27) + 1,
                                  0)
            done_now = lax.div(128 * t + 101, 27) + 1

            def rbody(r, c2):
                complete_row(r)
                return c2
            lax.fori_loop(done_prev, done_now, rbody, 0)
        return carry

    lax.fori_loop(0, 80, s_body, 0)

    ca, cb = tail_copies(0, 0)
    ca.start()
    cb.start()
    tail_runs = [(32 * (40 + v), 16 * v, 0) for v in range(5)] + \
        [(32 * 45, 73, 7)]

    def s5_body(px, carry):
        j = lax.rem(px, 2)

        @pl.when(px + 1 < 16)
        def _pref():
            ca, cb = tail_copies(px + 1, 1 - j)
            ca.start()
            cb.start()

        ca, cb = tail_copies(px, j)
        ca.wait()
        cb.wait()
        scatter_runs(px, j, tail_runs)
        return carry

    lax.fori_loop(0, 16, s5_body, 0)

    def rbody5(r, c2):
        complete_row(r)
        return c2
    lax.fori_loop(23, 28, rbody5, 0)

    pltpu.make_async_copy(stg.at[0], out_ref.at[b, 0], so.at[0]).wait()
    pltpu.make_async_copy(stg.at[1], out_ref.at[b, 0], so.at[1]).wait()


def kernel(x):
    xt = jnp.transpose(x, (0, 2, 3, 4, 1))
    xtail = jnp.pad(xt[:, :, :, :, 640:], ((0, 0),) * 4 + ((0, 39),))
    tbl = jnp.asarray(
        np.stack([np.stack([_BASE_A[i], _BASE_B[i]])
                  for i in range(len(_SPECS))]).reshape(-1))
    out = _overlap_add_sc(xt, xtail, tbl)
    return jnp.transpose(out, (0, 1, 3, 2))


# speedup vs baseline: 7.4976x; 1.0093x over previous
import functools

import numpy as np

import jax
import jax.numpy as jnp
from jax import lax
from jax.experimental import pallas as pl
from jax.experimental.pallas import tpu as pltpu
from jax.experimental.pallas import tpu_sc as plsc

_H = 224
_W = 224
_B = 4
_C = 32
_NY = 27
_NX = 27
_NP = _NY * _NX
_NSLOT = 8
_NTILE = 6
_QSTR = _C * 8 + 1
_SLOTSZ = 7200


def _run_specs():
    specs = []
    for t in range(5):
        for v in range(8):
            specs.append((t, 16 * v, 0))
    for v in range(5):
        specs.append((5, 16 * v, 0))
    specs.append((5, 73, 7))
    return specs


_SPECS = _run_specs()


def _base_tables():
    a, bb = [], []
    for (t, col0, mfrom) in _SPECS:
        p = 128 * t + col0 + np.arange(16)
        p = np.minimum(p, _NP - 1)
        iy = p // _NX
        ix = p % _NX
        a.append(((iy % _NSLOT) * _SLOTSZ + _QSTR * ix).astype(np.int32))
        bb.append((((iy + 1) % _NSLOT) * _SLOTSZ + _QSTR * ix).astype(np.int32))
    return np.stack(a), np.stack(bb)


_BASE_A, _BASE_B = _base_tables()

_mesh = plsc.VectorSubcoreMesh(core_axis_name="c", subcore_axis_name="s")


@functools.partial(
    pl.kernel,
    out_type=jax.ShapeDtypeStruct((_B, _H, _C, _W), jnp.float32),
    mesh=_mesh,
    scratch_types=[
        pltpu.VMEM((2, 2, _C, 128), jnp.float32),
        pltpu.VMEM((_NSLOT * _SLOTSZ,), jnp.float32),
        pltpu.VMEM((2, _C, _W), jnp.float32),
        pltpu.VMEM((len(_SPECS) * 2 * 16,), jnp.int32),
        pltpu.SemaphoreType.DMA((2,)),
        pltpu.SemaphoreType.DMA((2,)),
        pltpu.SemaphoreType.DMA((2,)),
        pltpu.SemaphoreType.DMA,
    ],
    compiler_params=pltpu.CompilerParams(needs_layout_passes=False),
)
def _overlap_add_sc(xt_ref, xtail_ref, tbl_ref, out_ref, tb, acc, stg, tblv,
                    sa, sb, so, st):
    iota16 = lax.iota(jnp.int32, 16)
    patt = lax.div(iota16, 8) * _QSTR + lax.rem(iota16, 8)
    cid = lax.axis_index("c")
    sid = lax.axis_index("s")
    wid = cid * 16 + sid
    b = wid // 8
    py = wid % 8

    pltpu.make_async_copy(tbl_ref, tblv, st).start()

    @plsc.parallel_loop(0, _NSLOT * _SLOTSZ // 16, unroll=4)
    def _zbody(i):
        acc[pl.ds(i * 16, 16)] = jnp.zeros((16,), jnp.float32)

    pltpu.make_async_copy(tbl_ref, tblv, st).wait()

    def in_copies(s, j):
        t = lax.div(s, 16)
        px = lax.rem(s, 16)
        l0 = pl.multiple_of(128 * t, 128)
        srca = xt_ref.at[b, py, px, :, pl.ds(l0, 128)]
        srcb = xt_ref.at[b, py + 8, px, :, pl.ds(l0, 128)]
        ca = pltpu.make_async_copy(srca, tb.at[0, j], sa.at[j])
        cb = pltpu.make_async_copy(srcb, tb.at[1, j], sb.at[j])
        return ca, cb

    def tail_copies(px, j):
        ca = pltpu.make_async_copy(
            xtail_ref.at[b, py, px], tb.at[0, j], sa.at[j])
        cb = pltpu.make_async_copy(
            xtail_ref.at[b, py + 8, px], tb.at[1, j], sb.at[j])
        return ca, cb

    def complete_row(r):
        jo = lax.rem(r, 2)
        slot = lax.rem(r, _NSLOT)
        h = 8 * r + py
        sy = jnp.where(jnp.logical_and(r >= 1, r <= 26), 0.5, 1.0).astype(
            jnp.float32)
        mid = jnp.full((16,), sy * 0.5, jnp.float32)
        lo = jnp.where(iota16 < 8, sy, sy * 0.5).astype(jnp.float32)
        hi = jnp.where(iota16 < 8, sy * 0.5, sy).astype(jnp.float32)

        @pl.when(r >= 2)
        def _drain():
            pltpu.make_async_copy(stg.at[jo], out_ref.at[b, 0],
                                  so.at[jo]).wait()

        @plsc.parallel_loop(0, _C, unroll=2)
        def _crow(c):
            base = slot * _SLOTSZ + 8 * c
            for k in range(14):
                scale = lo if k == 0 else (hi if k == 13 else mid)
                idx = patt + jnp.full((16,), base + 2 * _QSTR * k, jnp.int32)
                v = plsc.load_gather(acc, [idx]) * scale
                stg[jo, c, pl.ds(16 * k, 16)] = v

        pltpu.make_async_copy(stg.at[jo], out_ref.at[b, h], so.at[jo]).start()

        @plsc.parallel_loop(0, _SLOTSZ // 16, unroll=4)
        def _zrow(i):
            acc[pl.ds(slot * _SLOTSZ + i * 16, 16)] = jnp.zeros(
                (16,), jnp.float32)

    def scatter_runs(px, j, runs):
        @plsc.parallel_loop(0, _C, unroll=2)
        def _c_body(c):
            off = _QSTR * lax.div(px, 8) + 8 * c + lax.rem(px, 8)
            for plane in range(2):
                for (tbl_ds, col0, mfrom) in runs:
                    tv = tblv[pl.ds(tbl_ds + 16 * plane, 16)]
                    idx = tv + jnp.full((16,), off, jnp.int32)
                    val = tb[plane, j, c, pl.ds(col0, 16)]
                    if mfrom:
                        plsc.addupdate_scatter(acc, [idx], val,
                                               mask=(iota16 >= mfrom))
                    else:
                        plsc.addupdate_scatter(acc, [idx], val)

    ca, cb = in_copies(0, 0)
    ca.start()
    cb.start()

    def s_body(s, carry):
        j = lax.rem(s, 2)

        @pl.when(s + 1 < 80)
        def _pref():
            ca, cb = in_copies(s + 1, 1 - j)
            ca.start()
            cb.start()

        ca, cb = in_copies(s, j)
        ca.wait()
        cb.wait()

        t = lax.div(s, 16)
        px = lax.rem(s, 16)
        runs = [(32 * (8 * t + v), 16 * v, 0) for v in range(8)]
        scatter_runs(px, j, runs)

        @pl.when(lax.rem(s, 16) == 15)
        def _complete():
            done_prev = jnp.where(t > 0, lax.div(128 * (t - 1) + 101, 27) + 1,
                                  0)
            done_now = lax.div(128 * t + 101, 27) + 1

            def rbody(r, c2):
                complete_row(r)
                return c2
            lax.fori_loop(done_prev, done_now, rbody, 0)
        return carry

    lax.fori_loop(0, 80, s_body, 0)

    ca, cb = tail_copies(0, 0)
    ca.start()
    cb.start()
    tail_runs = [(32 * (40 + v), 16 * v, 0) for v in range(5)] + \
        [(32 * 45, 73, 7)]

    def s5_body(px, carry):
        j = lax.rem(px, 2)

        @pl.when(px + 1 < 16)
        def _pref():
            ca, cb = tail_copies(px + 1, 1 - j)
            ca.start()
            cb.start()

        ca, cb = tail_copies(px, j)
        ca.wait()
        cb.wait()
        scatter_runs(px, j, tail_runs)
        return carry

    lax.fori_loop(0, 16, s5_body, 0)

    def rbody5(r, c2):
        complete_row(r)
        return c2
    lax.fori_loop(23, 28, rbody5, 0)

    pltpu.make_async_copy(stg.at[0], out_ref.at[b, 0], so.at[0]).wait()
    pltpu.make_async_copy(stg.at[1], out_ref.at[b, 0], so.at[1]).wait()


def kernel(x):
    xt = jnp.transpose(x, (0, 2, 3, 4, 1))
    xtail = jnp.pad(xt[:, :, :, :, 640:], ((0, 0),) * 4 + ((0, 39),))
    tbl = jnp.asarray(
        np.stack([np.stack([_BASE_A[i], _BASE_B[i]])
                  for i in range(len(_SPECS))]).reshape(-1))
    out = _overlap_add_sc(xt, xtail, tbl)
    return jnp.transpose(out, (0, 1, 3, 2))


# speedup vs baseline: 7.6018x; 1.0139x over previous
import functools

import numpy as np

import jax
import jax.numpy as jnp
from jax import lax
from jax.experimental import pallas as pl
from jax.experimental.pallas import tpu as pltpu
from jax.experimental.pallas import tpu_sc as plsc

_H = 224
_W = 224
_B = 4
_C = 32
_NY = 27
_NX = 27
_NP = _NY * _NX
_NSLOT = 8
_NTILE = 6
_QSTR = _C * 8 + 1
_SLOTSZ = 7200


def _run_specs():
    specs = []
    for t in range(5):
        for v in range(8):
            specs.append((t, 16 * v, 0))
    for v in range(5):
        specs.append((5, 16 * v, 0))
    specs.append((5, 73, 7))
    return specs


_SPECS = _run_specs()


def _base_tables():
    a, bb = [], []
    for (t, col0, mfrom) in _SPECS:
        p = 128 * t + col0 + np.arange(16)
        p = np.minimum(p, _NP - 1)
        iy = p // _NX
        ix = p % _NX
        a.append(((iy % _NSLOT) * _SLOTSZ + _QSTR * ix).astype(np.int32))
        bb.append((((iy + 1) % _NSLOT) * _SLOTSZ + _QSTR * ix).astype(np.int32))
    return np.stack(a), np.stack(bb)


_BASE_A, _BASE_B = _base_tables()

_mesh = plsc.VectorSubcoreMesh(core_axis_name="c", subcore_axis_name="s")


@functools.partial(
    pl.kernel,
    out_type=jax.ShapeDtypeStruct((_B, _H, _C, _W), jnp.float32),
    mesh=_mesh,
    scratch_types=[
        pltpu.VMEM((2, 4, _C, 128), jnp.float32),
        pltpu.VMEM((_NSLOT * _SLOTSZ,), jnp.float32),
        pltpu.VMEM((2, _C, _W), jnp.float32),
        pltpu.VMEM((len(_SPECS) * 2 * 16,), jnp.int32),
        pltpu.SemaphoreType.DMA((4,)),
        pltpu.SemaphoreType.DMA((4,)),
        pltpu.SemaphoreType.DMA((2,)),
        pltpu.SemaphoreType.DMA,
    ],
    compiler_params=pltpu.CompilerParams(needs_layout_passes=False),
)
def _overlap_add_sc(xt_ref, xtail_ref, tbl_ref, out_ref, tb, acc, stg, tblv,
                    sa, sb, so, st):
    iota16 = lax.iota(jnp.int32, 16)
    patt = lax.div(iota16, 8) * _QSTR + lax.rem(iota16, 8)
    cid = lax.axis_index("c")
    sid = lax.axis_index("s")
    wid = cid * 16 + sid
    b = wid // 8
    py = wid % 8

    pltpu.make_async_copy(tbl_ref, tblv, st).start()

    @plsc.parallel_loop(0, _NSLOT * _SLOTSZ // 16, unroll=4)
    def _zbody(i):
        acc[pl.ds(i * 16, 16)] = jnp.zeros((16,), jnp.float32)

    pltpu.make_async_copy(tbl_ref, tblv, st).wait()

    def in_copies(s, j):
        t = lax.div(s, 16)
        px = lax.rem(s, 16)
        l0 = pl.multiple_of(128 * t, 128)
        srca = xt_ref.at[b, py, px, :, pl.ds(l0, 128)]
        srcb = xt_ref.at[b, py + 8, px, :, pl.ds(l0, 128)]
        ca = pltpu.make_async_copy(srca, tb.at[0, j], sa.at[j])
        cb = pltpu.make_async_copy(srcb, tb.at[1, j], sb.at[j])
        return ca, cb

    def tail_copies(px, j):
        ca = pltpu.make_async_copy(
            xtail_ref.at[b, py, px], tb.at[0, j], sa.at[j])
        cb = pltpu.make_async_copy(
            xtail_ref.at[b, py + 8, px], tb.at[1, j], sb.at[j])
        return ca, cb

    def complete_row(r):
        jo = lax.rem(r, 2)
        slot = lax.rem(r, _NSLOT)
        h = 8 * r + py
        sy = jnp.where(jnp.logical_and(r >= 1, r <= 26), 0.5, 1.0).astype(
            jnp.float32)
        mid = jnp.full((16,), sy * 0.5, jnp.float32)
        lo = jnp.where(iota16 < 8, sy, sy * 0.5).astype(jnp.float32)
        hi = jnp.where(iota16 < 8, sy * 0.5, sy).astype(jnp.float32)

        @pl.when(r >= 2)
        def _drain():
            pltpu.make_async_copy(stg.at[jo], out_ref.at[b, 0],
                                  so.at[jo]).wait()

        @plsc.parallel_loop(0, _C, unroll=2)
        def _crow(c):
            base = slot * _SLOTSZ + 8 * c
            for k in range(14):
                scale = lo if k == 0 else (hi if k == 13 else mid)
                idx = patt + jnp.full((16,), base + 2 * _QSTR * k, jnp.int32)
                v = plsc.load_gather(acc, [idx]) * scale
                stg[jo, c, pl.ds(16 * k, 16)] = v

        pltpu.make_async_copy(stg.at[jo], out_ref.at[b, h], so.at[jo]).start()

        @plsc.parallel_loop(0, _SLOTSZ // 16, unroll=4)
        def _zrow(i):
            acc[pl.ds(slot * _SLOTSZ + i * 16, 16)] = jnp.zeros(
                (16,), jnp.float32)

    def scatter_runs(px, j, runs):
        @plsc.parallel_loop(0, _C, unroll=2)
        def _c_body(c):
            off = _QSTR * lax.div(px, 8) + 8 * c + lax.rem(px, 8)
            for plane in range(2):
                for (tbl_ds, col0, mfrom) in runs:
                    tv = tblv[pl.ds(tbl_ds + 16 * plane, 16)]
                    idx = tv + jnp.full((16,), off, jnp.int32)
                    val = tb[plane, j, c, pl.ds(col0, 16)]
                    if mfrom:
                        plsc.addupdate_scatter(acc, [idx], val,
                                               mask=(iota16 >= mfrom))
                    else:
                        plsc.addupdate_scatter(acc, [idx], val)

    for sp in range(2):
        ca, cb = in_copies(sp, sp)
        ca.start()
        cb.start()

    def s_body(s, carry):
        j = lax.rem(s, 4)

        @pl.when(s + 2 < 80)
        def _pref():
            ca, cb = in_copies(s + 2, lax.rem(s + 2, 4))
            ca.start()
            cb.start()

        ca, cb = in_copies(s, j)
        ca.wait()
        cb.wait()

        t = lax.div(s, 16)
        px = lax.rem(s, 16)
        runs = [(32 * (8 * t + v), 16 * v, 0) for v in range(8)]
        scatter_runs(px, j, runs)

        @pl.when(lax.rem(s, 16) == 15)
        def _complete():
            done_prev = jnp.where(t > 0, lax.div(128 * (t - 1) + 101, 27) + 1,
                                  0)
            done_now = lax.div(128 * t + 101, 27) + 1

            def rbody(r, c2):
                complete_row(r)
                return c2
            lax.fori_loop(done_prev, done_now, rbody, 0)
        return carry

    lax.fori_loop(0, 80, s_body, 0)

    for sp in range(2):
        ca, cb = tail_copies(sp, sp)
        ca.start()
        cb.start()
    tail_runs = [(32 * (40 + v), 16 * v, 0) for v in range(5)] + \
        [(32 * 45, 73, 7)]

    def s5_body(px, carry):
        j = lax.rem(px, 4)

        @pl.when(px + 2 < 16)
        def _pref():
            ca, cb = tail_copies(px + 2, lax.rem(px + 2, 4))
            ca.start()
            cb.start()

        ca, cb = tail_copies(px, j)
        ca.wait()
        cb.wait()
        scatter_runs(px, j, tail_runs)
        return carry

    lax.fori_loop(0, 16, s5_body, 0)

    def rbody5(r, c2):
        complete_row(r)
        return c2
    lax.fori_loop(23, 28, rbody5, 0)

    pltpu.make_async_copy(stg.at[0], out_ref.at[b, 0], so.at[0]).wait()
    pltpu.make_async_copy(stg.at[1], out_ref.at[b, 0], so.at[1]).wait()


def kernel(x):
    xt = jnp.transpose(x, (0, 2, 3, 4, 1))
    xtail = jnp.pad(xt[:, :, :, :, 640:], ((0, 0),) * 4 + ((0, 39),))
    tbl = jnp.asarray(
        np.stack([np.stack([_BASE_A[i], _BASE_B[i]])
                  for i in range(len(_SPECS))]).reshape(-1))
    out = _overlap_add_sc(xt, xtail, tbl)
    return jnp.transpose(out, (0, 1, 3, 2))
